# Initial kernel scaffold; baseline (speedup 1.0000x reference)
import jax
import jax.numpy as jnp
from jax.experimental import pallas as pl


def kernel(feats, batch_ids, weight, bias):
    raise NotImplementedError("write your pallas kernel here")


# speedup vs baseline: 3.3319x; 3.3319x over previous
import functools

import jax
import jax.numpy as jnp
from jax import lax
from jax.experimental import pallas as pl
from jax.experimental.pallas import tpu as pltpu
from jax.experimental.pallas import tpu_sc as plsc

_NSEG = 8
_NTOK = 16384
_NFEAT = 256
_L = 16
_NC = 2
_NS = 16
_NW = _NC * _NS
_TPW = _NTOK // _NW
_CHUNK = 128
_NCH = _TPW // _CHUNK
_NGRP = _CHUNK // _L
_NCOL = _NFEAT // _L
_SROW = 2 * _NFEAT
_PROWS = _NSEG + 1

_mesh = plsc.VectorSubcoreMesh(
    core_axis_name="c", subcore_axis_name="s", num_cores=_NC
)


def _tree_sum(xs):
    while len(xs) > 1:
        xs = [a + b for a, b in zip(xs[::2], xs[1::2])]
    return xs[0]


def _splat16(v, s):
    idx = jnp.full((_L,), s, jnp.int32)
    dnums = lax.GatherDimensionNumbers(
        offset_dims=(), collapsed_slice_dims=(0,), start_index_map=(0,)
    )
    return lax.gather(
        v, idx[:, None], dnums, (1,),
        mode=lax.GatherScatterMode.PROMISE_IN_BOUNDS,
    )


def _rsqrt16(v):
    y = jnp.full((_L,), jnp.float32(2.0 ** 14))
    for k in range(-14, 9):
        y = jnp.where(v > jnp.float32(4.0 ** k), y * jnp.float32(0.5), y)
    for _ in range(6):
        y = y * (1.5 - 0.5 * v * y * y)
    return y


@functools.partial(
    pl.kernel,
    out_type=jax.ShapeDtypeStruct((_NC, _PROWS, _SROW), jnp.float32),
    mesh=_mesh,
    scratch_types=[
        pltpu.VMEM((_CHUNK, _NFEAT), jnp.float32),
        pltpu.VMEM((_PROWS, _SROW), jnp.float32),
        pltpu.VMEM((_PROWS, _SROW), jnp.float32),
        pltpu.VMEM((_TPW,), jnp.int32),
        pltpu.VMEM_SHARED((_NS, _PROWS, _SROW), jnp.float32),
    ],
)
def _stats(feats_hbm, ids_hbm, out_hbm, buf, acc, tmp, ids_v, shared):
    cid = lax.axis_index("c")
    sid = lax.axis_index("s")
    wid = sid * _NC + cid
    base = wid * _TPW
    lane = lax.iota(jnp.int32, _L)
    zero = jnp.zeros((_L,), jnp.float32)

    def zero_row(r, carry):
        for c in range(_SROW // _L):
            acc[r, pl.ds(c * _L, _L)] = zero
        return carry

    lax.fori_loop(0, _PROWS, zero_row, 0)

    pltpu.sync_copy(ids_hbm.at[pl.ds(base, _TPW)], ids_v)

    def chunk_body(ch, carry):
        pltpu.sync_copy(feats_hbm.at[pl.ds(base + ch * _CHUNK, _CHUNK)], buf)

        def grp_body(g, carry2):
            row0 = g * _L
            ids_g = ids_v[pl.ds(ch * _CHUNK + g * _L, _L)]
            smin = ids_g[0]
            smax = ids_g[_L - 1]

            @pl.when(smin == smax)
            def _():
                for c in range(_NCOL):
                    sl = pl.ds(c * _L, _L)
                    xs = [buf[row0 + t, sl] for t in range(_L)]
                    plsc.addupdate(acc.at[smin, sl], _tree_sum(xs))
                    plsc.addupdate(
                        acc.at[smin, pl.ds(_NFEAT + c * _L, _L)],
                        _tree_sum([x * x for x in xs]),
                    )
                plsc.addupdate(
                    acc.at[_NSEG, pl.ds(0, _L)],
                    jnp.where(lane == smin, jnp.float32(_L), jnp.float32(0)),
                )

            @pl.when(smin != smax)
            def _():
                for t in range(_L):
                    s_t = ids_g[t]
                    for c in range(_NCOL):
                        x = buf[row0 + t, pl.ds(c * _L, _L)]
                        plsc.addupdate(acc.at[s_t, pl.ds(c * _L, _L)], x)
                        plsc.addupdate(
                            acc.at[s_t, pl.ds(_NFEAT + c * _L, _L)], x * x
                        )
                    plsc.addupdate(
                        acc.at[_NSEG, pl.ds(0, _L)],
                        jnp.where(lane == s_t, jnp.float32(1), jnp.float32(0)),
                    )

            return carry2

        lax.fori_loop(0, _NGRP, grp_body, 0)
        return carry

    lax.fori_loop(0, _NCH, chunk_body, 0)

    pltpu.sync_copy(acc, shared.at[sid])
    plsc.subcore_barrier()
    for step in (8, 4, 2, 1):
        @pl.when(sid < step)
        def _():
            pltpu.sync_copy(shared.at[sid + step], tmp)

            def add_row(r, carry):
                for c in range(_SROW // _L):
                    sl = pl.ds(c * _L, _L)
                    acc[r, sl] = acc[r, sl] + tmp[r, sl]
                return carry

            lax.fori_loop(0, _PROWS, add_row, 0)
            pltpu.sync_copy(acc, shared.at[sid])

        plsc.subcore_barrier()

    @pl.when(sid == 0)
    def _():
        pltpu.sync_copy(acc, out_hbm.at[cid])


@functools.partial(
    pl.kernel,
    out_type=jax.ShapeDtypeStruct((_NTOK, _NFEAT), jnp.float32),
    mesh=_mesh,
    scratch_types=[
        pltpu.VMEM((_CHUNK, _NFEAT), jnp.float32),
        pltpu.VMEM((_PROWS, _SROW), jnp.float32),
        pltpu.VMEM((_PROWS, _SROW), jnp.float32),
        pltpu.VMEM((_NSEG, _NFEAT), jnp.float32),
        pltpu.VMEM((_NSEG, _NFEAT), jnp.float32),
        pltpu.VMEM((_TPW,), jnp.int32),
        pltpu.VMEM((1, _NFEAT), jnp.float32),
        pltpu.VMEM((1, _NFEAT), jnp.float32),
    ],
)
def _norm(
    feats_hbm, ids_hbm, part_hbm, w_hbm, b_hbm, out_hbm,
    buf, tot, tmp, a_v, b2_v, ids_v, w_v, bias_v,
):
    cid = lax.axis_index("c")
    sid = lax.axis_index("s")
    wid = sid * _NC + cid
    base = wid * _TPW
    lane = lax.iota(jnp.int32, _L)

    pltpu.sync_copy(ids_hbm.at[pl.ds(base, _TPW)], ids_v)
    pltpu.sync_copy(w_hbm, w_v)
    pltpu.sync_copy(b_hbm, bias_v)
    pltpu.sync_copy(part_hbm.at[0], tot)
    pltpu.sync_copy(part_hbm.at[1], tmp)

    def add_row(r, carry):
        for c in range(_SROW // _L):
            sl = pl.ds(c * _L, _L)
            tot[r, sl] = tot[r, sl] + tmp[r, sl]
        return carry

    lax.fori_loop(0, _PROWS, add_row, 0)

    cnt = jnp.maximum(tot[_NSEG, pl.ds(0, _L)], 1.0)
    rcv = 1.0 / cnt

    def seg_body(s, carry):
        rc = _splat16(rcv, s)
        for c in range(_NCOL):
            sl = pl.ds(c * _L, _L)
            sm = tot[s, sl]
            sq = tot[s, pl.ds(_NFEAT + c * _L, _L)]
            m = sm * rc
            var = jnp.maximum(sq * rc - m * m, 0.0) + jnp.float32(1e-8)
            a = _rsqrt16(var) * w_v[0, sl]
            a_v[s, sl] = a
            b2_v[s, sl] = bias_v[0, sl] - m * a
        return carry

    lax.fori_loop(0, _NSEG, seg_body, 0)

    def chunk_body(ch, carry):
        tok0 = base + ch * _CHUNK
        pltpu.sync_copy(feats_hbm.at[pl.ds(tok0, _CHUNK)], buf)

        def grp_body(g, carry2):
            row0 = g * _L
            ids_g = ids_v[pl.ds(ch * _CHUNK + g * _L, _L)]
            smin = ids_g[0]
            smax = ids_g[_L - 1]

            @pl.when(smin == smax)
            def _():
                for c in range(_NCOL):
                    sl = pl.ds(c * _L, _L)
                    a = a_v[smin, sl]
                    b = b2_v[smin, sl]
                    for t in range(_L):
                        buf[row0 + t, sl] = buf[row0 + t, sl] * a + b

            @pl.when(smin != smax)
            def _():
                for t in range(_L):
                    s_t = ids_g[t]
                    for c in range(_NCOL):
                        sl = pl.ds(c * _L, _L)
                        buf[row0 + t, sl] = (
                            buf[row0 + t, sl] * a_v[s_t, sl] + b2_v[s_t, sl]
                        )

            return carry2

        lax.fori_loop(0, _NGRP, grp_body, 0)
        pltpu.sync_copy(buf, out_hbm.at[pl.ds(tok0, _CHUNK)])
        return carry

    lax.fori_loop(0, _NCH, chunk_body, 0)


def kernel(feats, batch_ids, weight, bias):
    part = _stats(feats, batch_ids)
    return _norm(feats, batch_ids, part, weight, bias)


# speedup vs baseline: 3.5105x; 1.0536x over previous
import functools

import jax
import jax.numpy as jnp
from jax import lax
from jax.experimental import pallas as pl
from jax.experimental.pallas import tpu as pltpu
from jax.experimental.pallas import tpu_sc as plsc

_NSEG = 8
_NTOK = 16384
_NFEAT = 256
_L = 16
_NC = 2
_NS = 16
_NW = _NC * _NS
_TPW = _NTOK // _NW
_NCOL = _NFEAT // _L
_SROW = 2 * _NFEAT
_PROWS = _NSEG + 1

_CHUNK_S = 128
_NCH_S = _TPW // _CHUNK_S
_CHUNK_N = 64
_NCH_N = _TPW // _CHUNK_N

_mesh = plsc.VectorSubcoreMesh(
    core_axis_name="c", subcore_axis_name="s", num_cores=_NC
)


def _tree_sum(xs):
    while len(xs) > 1:
        xs = [a + b for a, b in zip(xs[::2], xs[1::2])]
    return xs[0]


def _splat16(v, s):
    idx = jnp.full((_L,), s, jnp.int32)
    dnums = lax.GatherDimensionNumbers(
        offset_dims=(), collapsed_slice_dims=(0,), start_index_map=(0,)
    )
    return lax.gather(
        v, idx[:, None], dnums, (1,),
        mode=lax.GatherScatterMode.PROMISE_IN_BOUNDS,
    )


def _rsqrt16(v):
    y = jnp.full((_L,), jnp.float32(2.0 ** 14))
    for k in range(-14, 9):
        y = jnp.where(v > jnp.float32(4.0 ** k), y * jnp.float32(0.5), y)
    for _ in range(6):
        y = y * (1.5 - 0.5 * v * y * y)
    return y


@functools.partial(
    pl.kernel,
    out_type=jax.ShapeDtypeStruct((_NC, _PROWS, _SROW), jnp.float32),
    mesh=_mesh,
    scratch_types=[
        pltpu.VMEM((_CHUNK_S, _NFEAT), jnp.float32),
        pltpu.VMEM((_CHUNK_S, _NFEAT), jnp.float32),
        pltpu.VMEM((_PROWS, _SROW), jnp.float32),
        pltpu.VMEM((_PROWS, _SROW), jnp.float32),
        pltpu.VMEM((_TPW,), jnp.int32),
        pltpu.VMEM_SHARED((_NS, _PROWS, _SROW), jnp.float32),
        pltpu.SemaphoreType.DMA,
        pltpu.SemaphoreType.DMA,
    ],
)
def _stats(feats_hbm, ids_hbm, out_hbm, buf0, buf1, acc, tmp, ids_v, shared,
           sem0, sem1):
    cid = lax.axis_index("c")
    sid = lax.axis_index("s")
    wid = sid * _NC + cid
    base = wid * _TPW
    lane = lax.iota(jnp.int32, _L)
    zero = jnp.zeros((_L,), jnp.float32)
    bufs = (buf0, buf1)
    sems = (sem0, sem1)

    def zero_row(r, carry):
        for c in range(_SROW // _L):
            acc[r, pl.ds(c * _L, _L)] = zero
        return carry

    lax.fori_loop(0, _PROWS, zero_row, 0)

    pltpu.sync_copy(ids_hbm.at[pl.ds(base, _TPW)], ids_v)

    def accum_chunk(ch, buf):
        def grp_body(g, carry2):
            row0 = g * _L
            ids_g = ids_v[pl.ds(ch * _CHUNK_S + g * _L, _L)]
            smin = ids_g[0]
            smax = ids_g[_L - 1]

            @pl.when(smin == smax)
            def _():
                for c in range(_NCOL):
                    sl = pl.ds(c * _L, _L)
                    xs = [buf[row0 + t, sl] for t in range(_L)]
                    plsc.addupdate(acc.at[smin, sl], _tree_sum(xs))
                    plsc.addupdate(
                        acc.at[smin, pl.ds(_NFEAT + c * _L, _L)],
                        _tree_sum([x * x for x in xs]),
                    )
                plsc.addupdate(
                    acc.at[_NSEG, pl.ds(0, _L)],
                    jnp.where(lane == smin, jnp.float32(_L), jnp.float32(0)),
                )

            @pl.when(smin != smax)
            def _():
                for t in range(_L):
                    s_t = ids_g[t]
                    for c in range(_NCOL):
                        x = buf[row0 + t, pl.ds(c * _L, _L)]
                        plsc.addupdate(acc.at[s_t, pl.ds(c * _L, _L)], x)
                        plsc.addupdate(
                            acc.at[s_t, pl.ds(_NFEAT + c * _L, _L)], x * x
                        )
                    plsc.addupdate(
                        acc.at[_NSEG, pl.ds(0, _L)],
                        jnp.where(lane == s_t, jnp.float32(1), jnp.float32(0)),
                    )

            return carry2

        lax.fori_loop(0, _CHUNK_S // _L, grp_body, 0)

    for ch in range(2):
        pltpu.async_copy(
            feats_hbm.at[pl.ds(base + ch * _CHUNK_S, _CHUNK_S)],
            bufs[ch], sems[ch],
        )

    def pair_body(p, carry):
        for b in range(2):
            ch = 2 * p + b
            pltpu.make_async_copy(
                feats_hbm.at[pl.ds(base + ch * _CHUNK_S, _CHUNK_S)],
                bufs[b], sems[b],
            ).wait()
            accum_chunk(ch, bufs[b])

            @pl.when(ch + 2 < _NCH_S)
            def _():
                pltpu.async_copy(
                    feats_hbm.at[pl.ds(base + (ch + 2) * _CHUNK_S, _CHUNK_S)],
                    bufs[b], sems[b],
                )

        return carry

    lax.fori_loop(0, _NCH_S // 2, pair_body, 0)

    pltpu.sync_copy(acc, shared.at[sid])
    plsc.subcore_barrier()
    for step in (8, 4, 2, 1):
        @pl.when(sid < step)
        def _():
            pltpu.sync_copy(shared.at[sid + step], tmp)

            def add_row(r, carry):
                for c in range(_SROW // _L):
                    sl = pl.ds(c * _L, _L)
                    acc[r, sl] = acc[r, sl] + tmp[r, sl]
                return carry

            lax.fori_loop(0, _PROWS, add_row, 0)
            pltpu.sync_copy(acc, shared.at[sid])

        plsc.subcore_barrier()

    @pl.when(sid == 0)
    def _():
        pltpu.sync_copy(acc, out_hbm.at[cid])


@functools.partial(
    pl.kernel,
    out_type=jax.ShapeDtypeStruct((_NTOK, _NFEAT), jnp.float32),
    mesh=_mesh,
    scratch_types=[
        pltpu.VMEM((_CHUNK_N, _NFEAT), jnp.float32),
        pltpu.VMEM((_CHUNK_N, _NFEAT), jnp.float32),
        pltpu.VMEM((_CHUNK_N, _NFEAT), jnp.float32),
        pltpu.VMEM((_CHUNK_N, _NFEAT), jnp.float32),
        pltpu.VMEM((_PROWS, _SROW), jnp.float32),
        pltpu.VMEM((_PROWS, _SROW), jnp.float32),
        pltpu.VMEM((_NSEG, _NFEAT), jnp.float32),
        pltpu.VMEM((_NSEG, _NFEAT), jnp.float32),
        pltpu.VMEM((_TPW,), jnp.int32),
        pltpu.VMEM((1, _NFEAT), jnp.float32),
        pltpu.VMEM((1, _NFEAT), jnp.float32),
        pltpu.SemaphoreType.DMA,
        pltpu.SemaphoreType.DMA,
        pltpu.SemaphoreType.DMA,
        pltpu.SemaphoreType.DMA,
    ],
)
def _norm(
    feats_hbm, ids_hbm, part_hbm, w_hbm, b_hbm, out_hbm,
    ibuf0, ibuf1, obuf0, obuf1, tot, tmp, a_v, b2_v, ids_v, w_v, bias_v,
    sem_i0, sem_i1, sem_o0, sem_o1,
):
    cid = lax.axis_index("c")
    sid = lax.axis_index("s")
    wid = sid * _NC + cid
    base = wid * _TPW
    ibufs = (ibuf0, ibuf1)
    obufs = (obuf0, obuf1)
    sems_i = (sem_i0, sem_i1)
    sems_o = (sem_o0, sem_o1)

    pltpu.sync_copy(ids_hbm.at[pl.ds(base, _TPW)], ids_v)

    for ch in range(2):
        pltpu.async_copy(
            feats_hbm.at[pl.ds(base + ch * _CHUNK_N, _CHUNK_N)],
            ibufs[ch], sems_i[ch],
        )

    pltpu.sync_copy(w_hbm, w_v)
    pltpu.sync_copy(b_hbm, bias_v)
    pltpu.sync_copy(part_hbm.at[0], tot)
    pltpu.sync_copy(part_hbm.at[1], tmp)

    def add_row(r, carry):
        for c in range(_SROW // _L):
            sl = pl.ds(c * _L, _L)
            tot[r, sl] = tot[r, sl] + tmp[r, sl]
        return carry

    lax.fori_loop(0, _PROWS, add_row, 0)

    cnt = jnp.maximum(tot[_NSEG, pl.ds(0, _L)], 1.0)
    rcv = 1.0 / cnt

    def seg_body(s, carry):
        rc = _splat16(rcv, s)
        for c in range(_NCOL):
            sl = pl.ds(c * _L, _L)
            sm = tot[s, sl]
            sq = tot[s, pl.ds(_NFEAT + c * _L, _L)]
            m = sm * rc
            var = jnp.maximum(sq * rc - m * m, 0.0) + jnp.float32(1e-8)
            a = _rsqrt16(var) * w_v[0, sl]
            a_v[s, sl] = a
            b2_v[s, sl] = bias_v[0, sl] - m * a
        return carry

    lax.fori_loop(0, _NSEG, seg_body, 0)

    def norm_chunk(ch, ibuf, obuf):
        def grp_body(g, carry2):
            row0 = g * _L
            ids_g = ids_v[pl.ds(ch * _CHUNK_N + g * _L, _L)]
            smin = ids_g[0]
            smax = ids_g[_L - 1]

            @pl.when(smin == smax)
            def _():
                for c in range(_NCOL):
                    sl = pl.ds(c * _L, _L)
                    a = a_v[smin, sl]
                    b = b2_v[smin, sl]
                    for t in range(_L):
                        obuf[row0 + t, sl] = ibuf[row0 + t, sl] * a + b

            @pl.when(smin != smax)
            def _():
                for t in range(_L):
                    s_t = ids_g[t]
                    for c in range(_NCOL):
                        sl = pl.ds(c * _L, _L)
                        obuf[row0 + t, sl] = (
                            ibuf[row0 + t, sl] * a_v[s_t, sl] + b2_v[s_t, sl]
                        )

            return carry2

        lax.fori_loop(0, _CHUNK_N // _L, grp_body, 0)

    def pair_body(p, carry):
        for b in range(2):
            ch = 2 * p + b
            pltpu.make_async_copy(
                feats_hbm.at[pl.ds(base + ch * _CHUNK_N, _CHUNK_N)],
                ibufs[b], sems_i[b],
            ).wait()

            @pl.when(p > 0)
            def _():
                pltpu.make_async_copy(
                    obufs[b],
                    out_hbm.at[pl.ds(base + (ch - 2) * _CHUNK_N, _CHUNK_N)],
                    sems_o[b],
                ).wait()

            norm_chunk(ch, ibufs[b], obufs[b])
            pltpu.async_copy(
                obufs[b],
                out_hbm.at[pl.ds(base + ch * _CHUNK_N, _CHUNK_N)],
                sems_o[b],
            )

            @pl.when(ch + 2 < _NCH_N)
            def _():
                pltpu.async_copy(
                    feats_hbm.at[pl.ds(base + (ch + 2) * _CHUNK_N, _CHUNK_N)],
                    ibufs[b], sems_i[b],
                )

        return carry

    lax.fori_loop(0, _NCH_N // 2, pair_body, 0)
    for b in range(2):
        ch = _NCH_N - 2 + b
        pltpu.make_async_copy(
            obufs[b],
            out_hbm.at[pl.ds(base + ch * _CHUNK_N, _CHUNK_N)],
            sems_o[b],
        ).wait()


def kernel(feats, batch_ids, weight, bias):
    part = _stats(feats, batch_ids)
    return _norm(feats, batch_ids, part, weight, bias)
